# baseline (speedup 1.0000x reference)
import functools

import jax
import jax.numpy as jnp
from jax import lax
from jax.experimental import pallas as pl
from jax.experimental.pallas import tpu as pltpu
from jax.experimental.pallas import tpu_sc as plsc

N = 10000
E = 160000
NE = 10
C = 128
SH = 4
NB = 8
RMAX = 5.0
G = 4
AVG = 16.0
D = SH * C

NC = 2
NS = 16
NW = NC * NS
IB = 128
GB = 40
E_PAD = NW * GB * IB
SB = E_PAD // NS // IB
TROW = 624
NROWS = 640

BE = 2048
EGRID = E_PAD // BE
BN = 2000
NGRID = N // BN


def _silu(x):
    return x * jax.nn.sigmoid(x)


def _node_pre_body(na, we, wup1, h0_o, hup1_o):
    h0 = jnp.dot(na[...], we[...], preferred_element_type=jnp.float32)
    h0_o[...] = h0
    hup1_o[...] = jnp.dot(h0, wup1[...], preferred_element_type=jnp.float32)


def _edge1_body(pos8, shT, hup1s, r10, r11, r12, r13,
                msg1_o, efT_o, yT_o):
    pp = pos8[...]
    st = shT[...]
    vx = pp[0:1] - pp[3:4] + st[0:1]
    vy = pp[1:2] - pp[4:5] + st[1:2]
    vz = pp[2:3] - pp[5:6] + st[2:3]
    r = jnp.sqrt(vx * vx + vy * vy + vz * vz)
    rinv = 1.0 / (r + 1e-9)
    s3 = 3.0 ** 0.5
    yT = jnp.concatenate([jnp.ones_like(r), s3 * vx * rinv,
                          s3 * vy * rinv, s3 * vz * rinv], axis=0)
    yT_o[...] = yT
    k = (lax.broadcasted_iota(jnp.int32, (NB, 1), 0) + 1).astype(jnp.float32)
    bes = jnp.sqrt(2.0 / RMAX) * jnp.sin(k * (jnp.pi / RMAX) * r) * rinv
    x = r / RMAX
    x2 = x * x
    x4 = x2 * x2
    x5 = x4 * x
    x6 = x5 * x
    x7 = x6 * x
    p = 5.0
    fcut = (1.0 - ((p + 1) * (p + 2) / 2.0) * x5 + p * (p + 2) * x6
            - (p * (p + 1) / 2.0) * x7)
    fcut = jnp.where(x < 1.0, fcut, 0.0)
    ef = bes * fcut
    efT_o[...] = ef

    r1 = _radialT(ef, r10, r11, r12, r13)
    y = _cols4(yT)
    hs = hup1s[...]
    rows = []
    for m in range(SH):
        rl = r1[:, :C] if m == 0 else r1[:, C:]
        rows.append((rl * y[:, m:m + 1] * hs)[None])
    msg1_o[...] = jnp.concatenate(rows, axis=0)


def _radialT(efT, w0, w1, w2, w3):
    h = _silu(lax.dot_general(efT, w0[...], (((0,), (0,)), ((), ())),
                              preferred_element_type=jnp.float32))
    h = _silu(jnp.dot(h, w1[...], preferred_element_type=jnp.float32))
    h = _silu(jnp.dot(h, w2[...], preferred_element_type=jnp.float32))
    return jnp.dot(h, w3[...], preferred_element_type=jnp.float32)


def _cols4(yT):
    i0 = lax.broadcasted_iota(jnp.int32, (4, 4), 0)
    i1 = lax.broadcasted_iota(jnp.int32, (4, 4), 1)
    eye4 = jnp.where(i0 == i1, 1.0, 0.0).astype(jnp.float32)
    return lax.dot_general(yT, eye4, (((0,), (0,)), ((), ())),
                           preferred_element_type=jnp.float32)


def _edge2_body(efT_i, yT_i, hup2s, r20, r21, r22, r23, msg2_o):
    r2 = _radialT(efT_i[...], r20, r21, r22, r23)
    y = _cols4(yT_i[...])
    hs = hup2s[...]
    rows = []
    for m in range(SH):
        rl = r2[:, :C] if m == 0 else r2[:, C:]
        rows.append((rl * y[:, m:m + 1] * hs)[None])
    msg2_o[...] = jnp.concatenate(rows, axis=0)


def _node_mid_body(agg1, wmix1, na, h0, pc1, wsc1, sc1a, wup2, wr1,
                   hup2_o, ne1_o):
    inv = 1.0 / AVG
    parts = []
    for m in range(SH):
        wl = wmix1[...][0] if m == 0 else wmix1[...][1]
        parts.append(jnp.dot(agg1[...][m] * inv, wl,
                             preferred_element_type=jnp.float32))
    m1 = jnp.concatenate(parts, axis=1)
    na_v = na[...]
    c0 = jnp.dot(na_v, pc1[...][0], preferred_element_type=jnp.float32)
    c1 = jnp.dot(na_v, pc1[...][1], preferred_element_type=jnp.float32)
    c2 = jnp.dot(na_v, pc1[...][2], preferred_element_type=jnp.float32)
    sc1 = (jnp.dot(h0[...], wsc1[...], preferred_element_type=jnp.float32)
           * jnp.dot(na_v, sc1a[...], preferred_element_type=jnp.float32))
    h1 = c0 * m1 + c1 * m1 * m1 + c2 * m1 * m1 * m1 + sc1
    hs = h1[:, :C]
    ne1_o[...] = jnp.dot(hs, wr1[...], preferred_element_type=jnp.float32)
    hup2_o[...] = jnp.dot(hs, wup2[...], preferred_element_type=jnp.float32)


def _node_out_body(agg2, wmix2, na, hup2, pc2, wout2, sc2a, nlw1, nlw2, ae,
                   ne1, nrg_o):
    inv = 1.0 / AVG
    parts = []
    for m in range(SH):
        wl = wmix2[...][0] if m == 0 else wmix2[...][1]
        parts.append(jnp.dot(agg2[...][m] * inv, wl,
                             preferred_element_type=jnp.float32))
    m2 = jnp.concatenate(parts, axis=1)
    na_v = na[...]
    c0 = jnp.dot(na_v, pc2[...][0], preferred_element_type=jnp.float32)
    c1 = jnp.dot(na_v, pc2[...][1], preferred_element_type=jnp.float32)
    c2 = jnp.dot(na_v, pc2[...][2], preferred_element_type=jnp.float32)
    p = c0 * m2 + c1 * m2 * m2 + c2 * m2 * m2 * m2
    h2 = (jnp.dot(p, wout2[...], preferred_element_type=jnp.float32)
          + hup2[...] * jnp.dot(na_v, sc2a[...],
                                preferred_element_type=jnp.float32))
    t = _silu(jnp.dot(h2, nlw1[...], preferred_element_type=jnp.float32))
    ne2 = jnp.dot(t, nlw2[...], preferred_element_type=jnp.float32)
    e0 = jnp.dot(na_v, ae[...], preferred_element_type=jnp.float32)
    nrg_o[...] = e0 + ne1[...] + ne2


def _graph_sum_body(ne, out):
    out[...] = jnp.sum(ne[...], axis=1)[None, :]


GB0 = 16
GB1 = 64


def _sc_gather3(px, py, pz, hup1, src2d, dst2d, pos8_o, hup1s_o,
                px_v, py_v, pz_v, idxs_all, idxd_all, rh0, rh1, pb0, pb1,
                sg0, sg1, sw0, sw1, sp0, sp1):
    c = lax.axis_index("c")
    s = lax.axis_index("s")
    pltpu.sync_copy(px, px_v)
    pltpu.sync_copy(py, py_v)
    pltpu.sync_copy(pz, pz_v)
    rh = (rh0, rh1)
    pb = (pb0, pb1)
    sg = (sg0, sg1)
    sw = (sw0, sw1)
    sp = (sp0, sp1)

    def run(row0, nb):
        pltpu.sync_copy(src2d.at[pl.ds(row0, nb)], idxs_all.at[pl.ds(0, nb)])
        pltpu.sync_copy(dst2d.at[pl.ds(row0, nb)], idxd_all.at[pl.ds(0, nb)])

        def fire_g(j, b):
            pltpu.async_copy(hup1.at[idxs_all.at[j]], rh[b], sg[b])

        def wait_g(b):
            pltpu.make_async_copy(hup1.at[idxs_all.at[0]], rh[b],
                                  sg[b]).wait()

        def pos_gather(j, b):
            def chunk(k, carry):
                sl = pl.ds(k * 16, 16)
                iv_s = idxs_all[j, sl]
                iv_d = idxd_all[j, sl]
                pb[b][0, sl] = plsc.load_gather(px_v, [iv_s])
                pb[b][1, sl] = plsc.load_gather(py_v, [iv_s])
                pb[b][2, sl] = plsc.load_gather(pz_v, [iv_s])
                pb[b][3, sl] = plsc.load_gather(px_v, [iv_d])
                pb[b][4, sl] = plsc.load_gather(py_v, [iv_d])
                pb[b][5, sl] = plsc.load_gather(pz_v, [iv_d])
                return carry

            lax.fori_loop(0, IB // 16, chunk, 0)

        def fire_w(j, b):
            base = (row0 + j) * IB
            pltpu.async_copy(rh[b], hup1s_o.at[pl.ds(base, IB)], sw[b])
            pltpu.async_copy(pb[b], pos8_o.at[:, pl.ds(base, IB)], sp[b])

        def wait_sw(b):
            pltpu.make_async_copy(rh[b], hup1s_o.at[pl.ds(0, IB)],
                                  sw[b]).wait()

        def wait_sp(b):
            pltpu.make_async_copy(pb[b], pos8_o.at[:, pl.ds(0, IB)],
                                  sp[b]).wait()

        fire_g(0, 0)

        def pair(gp, carry):
            for b in (0, 1):
                j = 2 * gp + b
                ob = 1 - b
                pl.when(j >= 2)(lambda: wait_sp(b))
                pos_gather(j, b)
                wait_g(b)
                pl.when(j >= 1)(lambda: wait_sw(ob))
                pl.when(j + 1 < nb)(lambda: fire_g(j + 1, ob))
                fire_w(j, b)
            return carry

        lax.fori_loop(0, nb // 2, pair, 0)
        wait_sw(1)
        wait_sp(0)
        wait_sp(1)

    pl.when(c == 0)(lambda: run(s * GB0, GB0))
    pl.when(c == 1)(lambda: run(NS * GB0 + s * GB1, GB1))


def _sc_gather1(hup2, src2d, hup2s_o, idxs_all, rh0, rh1, sg0, sg1,
                sw0, sw1):
    c = lax.axis_index("c")
    s = lax.axis_index("s")
    rh = (rh0, rh1)
    sg = (sg0, sg1)
    sw = (sw0, sw1)

    def run(row0, nb):
        pltpu.sync_copy(src2d.at[pl.ds(row0, nb)], idxs_all.at[pl.ds(0, nb)])

        def fire_g(j, b):
            pltpu.async_copy(hup2.at[idxs_all.at[j]], rh[b], sg[b])

        def wait_g(b):
            pltpu.make_async_copy(hup2.at[idxs_all.at[0]], rh[b],
                                  sg[b]).wait()

        def fire_w(j, b):
            pltpu.async_copy(rh[b], hup2s_o.at[pl.ds((row0 + j) * IB, IB)],
                             sw[b])

        def wait_w(b):
            pltpu.make_async_copy(rh[b], hup2s_o.at[pl.ds(0, IB)],
                                  sw[b]).wait()

        fire_g(0, 0)

        def pair(gp, carry):
            for b in (0, 1):
                j = 2 * gp + b
                ob = 1 - b
                wait_g(b)
                fire_w(j, b)
                pl.when(j >= 1)(lambda: wait_w(ob))
                pl.when(j + 1 < nb)(lambda: fire_g(j + 1, ob))
            return carry

        lax.fori_loop(0, nb // 2, pair, 0)
        wait_w(1)

    pl.when(c == 0)(lambda: run(s * GB0, GB0))
    pl.when(c == 1)(lambda: run(NS * GB0 + s * GB1, GB1))


def _sc_scatter(msg, dst2d, zeros, agg_o, agg_sp, idx_all, mr0, mr1,
                sm0, sm1, ss0, ss1):
    c = lax.axis_index("c")
    s = lax.axis_index("s")
    row0 = s * TROW
    pltpu.sync_copy(dst2d.at[pl.ds(s * SB, SB)], idx_all)
    mr = (mr0, mr1)
    sm = (sm0, sm1)
    ss = (ss0, ss1)
    for jc in range(2):
        m = 2 * c + jc

        def fire_msg(j, b):
            pltpu.async_copy(msg.at[m, pl.ds((s * SB + j) * IB, IB)],
                             mr[b], sm[b])

        def wait_msg(b):
            pltpu.make_async_copy(msg.at[m, pl.ds(0, IB)], mr[b],
                                  sm[b]).wait()

        def fire_sc(j, b):
            pltpu.async_copy(mr[b], agg_sp.at[idx_all.at[j]], ss[b],
                             add=True)

        def wait_sc(b):
            pltpu.make_async_copy(mr[b], agg_sp.at[idx_all.at[0]],
                                  ss[b]).wait()

        pltpu.sync_copy(zeros, agg_sp.at[pl.ds(row0, NROWS)])
        plsc.subcore_barrier()
        fire_msg(0, 0)

        def pair(gp, carry):
            for b in (0, 1):
                j = 2 * gp + b
                ob = 1 - b
                wait_msg(b)
                fire_sc(j, b)
                pl.when(j >= 1)(lambda: wait_sc(ob))
                pl.when(j + 1 < SB)(lambda: fire_msg(j + 1, ob))
            return carry

        lax.fori_loop(0, SB // 2, pair, 0)
        wait_sc(1)
        plsc.subcore_barrier()
        pltpu.sync_copy(agg_sp.at[pl.ds(row0, NROWS)],
                        agg_o.at[m, pl.ds(row0, NROWS)])


@functools.cache
def _sc_kernels():
    mesh = plsc.VectorSubcoreMesh(core_axis_name="c", subcore_axis_name="s",
                                  num_cores=NC, num_subcores=NS)
    gather3 = pl.kernel(
        _sc_gather3,
        out_type=[jax.ShapeDtypeStruct((8, E_PAD), jnp.float32),
                  jax.ShapeDtypeStruct((E_PAD, C), jnp.float32)],
        mesh=mesh,
        compiler_params=pltpu.CompilerParams(needs_layout_passes=False),
        scratch_types=[pltpu.VMEM((N,), jnp.float32),
                       pltpu.VMEM((N,), jnp.float32),
                       pltpu.VMEM((N,), jnp.float32),
                       pltpu.VMEM((GB1, IB), jnp.int32),
                       pltpu.VMEM((GB1, IB), jnp.int32),
                       pltpu.VMEM((IB, C), jnp.float32),
                       pltpu.VMEM((IB, C), jnp.float32),
                       pltpu.VMEM((8, IB), jnp.float32),
                       pltpu.VMEM((8, IB), jnp.float32),
                       pltpu.SemaphoreType.DMA,
                       pltpu.SemaphoreType.DMA,
                       pltpu.SemaphoreType.DMA,
                       pltpu.SemaphoreType.DMA,
                       pltpu.SemaphoreType.DMA,
                       pltpu.SemaphoreType.DMA],
    )
    gather1 = pl.kernel(
        _sc_gather1,
        out_type=[jax.ShapeDtypeStruct((E_PAD, C), jnp.float32)],
        mesh=mesh,
        scratch_types=[pltpu.VMEM((GB1, IB), jnp.int32),
                       pltpu.VMEM((IB, C), jnp.float32),
                       pltpu.VMEM((IB, C), jnp.float32),
                       pltpu.SemaphoreType.DMA,
                       pltpu.SemaphoreType.DMA,
                       pltpu.SemaphoreType.DMA,
                       pltpu.SemaphoreType.DMA],
    )
    scatter = pl.kernel(
        _sc_scatter,
        out_type=[jax.ShapeDtypeStruct((SH, N, C), jnp.float32)],
        mesh=mesh,
        scratch_types=[pltpu.VMEM_SHARED((N, C), jnp.float32),
                       pltpu.VMEM((SB, IB), jnp.int32),
                       pltpu.VMEM((IB, C), jnp.float32),
                       pltpu.VMEM((IB, C), jnp.float32),
                       pltpu.SemaphoreType.DMA,
                       pltpu.SemaphoreType.DMA,
                       pltpu.SemaphoreType.DMA,
                       pltpu.SemaphoreType.DMA],
    )
    return gather3, gather1, scatter


def _full(shape):
    return pl.BlockSpec(shape, lambda *_: tuple(0 for _ in shape))


def _node_pre(na, we, wup1):
    return pl.pallas_call(
        _node_pre_body,
        out_shape=[jax.ShapeDtypeStruct((N, C), jnp.float32),
                   jax.ShapeDtypeStruct((N, C), jnp.float32)],
    )(na, we, wup1)


def _edge1(pos8, shT, hup1s, rad1):
    eb = lambda w: pl.BlockSpec((BE, w), lambda i: (i, 0))
    return pl.pallas_call(
        _edge1_body,
        grid=(EGRID,),
        in_specs=[pl.BlockSpec((8, BE), lambda i: (0, i)),
                  pl.BlockSpec((4, BE), lambda i: (0, i)), eb(C),
                  _full((NB, 64)), _full((64, 64)), _full((64, 64)),
                  _full((64, 2 * C))],
        out_specs=[pl.BlockSpec((SH, BE, C), lambda i: (0, i, 0)),
                   pl.BlockSpec((NB, BE), lambda i: (0, i)),
                   pl.BlockSpec((4, BE), lambda i: (0, i))],
        out_shape=[jax.ShapeDtypeStruct((SH, E_PAD, C), jnp.float32),
                   jax.ShapeDtypeStruct((NB, E_PAD), jnp.float32),
                   jax.ShapeDtypeStruct((4, E_PAD), jnp.float32)],
    )(pos8, shT, hup1s, *rad1)


def _edge2(efT, yT, hup2s, rad2):
    eb = lambda w: pl.BlockSpec((BE, w), lambda i: (i, 0))
    return pl.pallas_call(
        _edge2_body,
        grid=(EGRID,),
        in_specs=[pl.BlockSpec((NB, BE), lambda i: (0, i)),
                  pl.BlockSpec((4, BE), lambda i: (0, i)), eb(C),
                  _full((NB, 64)), _full((64, 64)), _full((64, 64)),
                  _full((64, 2 * C))],
        out_specs=[pl.BlockSpec((SH, BE, C), lambda i: (0, i, 0))],
        out_shape=[jax.ShapeDtypeStruct((SH, E_PAD, C), jnp.float32)],
    )(efT, yT, hup2s, *rad2)


def _node_mid(agg1, wmix1, na, h0, pc1, wsc1, sc1a, wup2, wr1):
    nb = lambda w: pl.BlockSpec((BN, w), lambda i: (i, 0))
    return pl.pallas_call(
        _node_mid_body,
        grid=(NGRID,),
        in_specs=[pl.BlockSpec((SH, BN, C), lambda i: (0, i, 0)),
                  _full((2, C, C)), nb(NE), nb(C), _full((3, NE, D)),
                  _full((C, D)), _full((NE, 1)), _full((C, C)),
                  _full((C, 1))],
        out_specs=[nb(C), nb(1)],
        out_shape=[jax.ShapeDtypeStruct((N, C), jnp.float32),
                   jax.ShapeDtypeStruct((N, 1), jnp.float32)],
    )(agg1, wmix1, na, h0, pc1, wsc1, sc1a, wup2, wr1)


def _node_out(agg2, wmix2, na, hup2, pc2, wout2, sc2a, nlw1, nlw2, ae, ne1):
    nb = lambda w: pl.BlockSpec((BN, w), lambda i: (i, 0))
    return pl.pallas_call(
        _node_out_body,
        grid=(NGRID,),
        in_specs=[pl.BlockSpec((SH, BN, C), lambda i: (0, i, 0)),
                  _full((2, C, C)), nb(NE), nb(C), _full((3, NE, D)),
                  _full((D, C)), _full((NE, 1)), _full((C, 16)),
                  _full((16, 1)), _full((NE, 1)), nb(1)],
        out_specs=[nb(1)],
        out_shape=[jax.ShapeDtypeStruct((N, 1), jnp.float32)],
    )(agg2, wmix2, na, hup2, pc2, wout2, sc2a, nlw1, nlw2, ae, ne1)


def _graph_sum(ne_rows):
    return pl.pallas_call(
        _graph_sum_body,
        out_shape=jax.ShapeDtypeStruct((1, G), jnp.float32),
    )(ne_rows)


def kernel(positions, node_attrs, shifts, params, edge_index, batch):
    p = params
    src2d = jnp.pad(edge_index[0].astype(jnp.int32),
                    (0, E_PAD - E)).reshape(E_PAD // IB, IB)
    dst2d = jnp.pad(edge_index[1].astype(jnp.int32),
                    (0, E_PAD - E)).reshape(E_PAD // IB, IB)
    shT = jnp.pad(shifts.T, ((0, 1), (0, E_PAD - E)))
    px = jnp.asarray(positions[:, 0])
    py = jnp.asarray(positions[:, 1])
    pz = jnp.asarray(positions[:, 2])
    zeros = jnp.zeros((NROWS, C), jnp.float32)

    gather3, gather1, scatter = _sc_kernels()
    h0, hup1 = _node_pre(node_attrs, p['W_embed'], p['W_up1'])
    pos8, hup1s = gather3(px, py, pz, hup1, src2d, dst2d)
    msg1, efT, yT = _edge1(pos8, shT, hup1s, p['rad1'])
    agg1, = scatter(msg1, dst2d, zeros)
    hup2, ne1 = _node_mid(agg1, p['W_mix1'], node_attrs, h0, p['prod1_c'],
                          p['W_sc1'], p['sc1_a'][:, None], p['W_up2'],
                          p['w_r1'])
    hup2s, = gather1(hup2, src2d)
    msg2, = _edge2(efT, yT, hup2s, p['rad2'])
    agg2, = scatter(msg2, dst2d, zeros)
    ne, = _node_out(agg2, p['W_mix2'], node_attrs, hup2, p['prod2_c'],
                   p['W_out2'], p['sc2_a'][:, None], p['nl_w1'], p['nl_w2'],
                   p['atomic_energies'][:, None], ne1)
    return _graph_sum(ne.reshape(G, N // G)).reshape(G)


# speedup vs baseline: 1.0163x; 1.0163x over previous
import functools

import jax
import jax.numpy as jnp
from jax import lax
from jax.experimental import pallas as pl
from jax.experimental.pallas import tpu as pltpu
from jax.experimental.pallas import tpu_sc as plsc

N = 10000
E = 160000
NE = 10
C = 128
SH = 4
NB = 8
RMAX = 5.0
G = 4
AVG = 16.0
D = SH * C

NC = 2
NS = 16
NW = NC * NS
IB = 128
GB = 40
E_PAD = NW * GB * IB
SB = E_PAD // NS // IB
TROW = 624
NROWS = 640

BE = 2048
EGRID = E_PAD // BE
BN = 2000
NGRID = N // BN


def _silu(x):
    return x * jax.nn.sigmoid(x)


def _node_pre_body(na, we, wup1, h0_o, hup1_o):
    h0 = jnp.dot(na[...], we[...], preferred_element_type=jnp.float32)
    h0_o[...] = h0
    hup1_o[...] = jnp.dot(h0, wup1[...], preferred_element_type=jnp.float32)


def _edge1_body(pos8, shT, hup1s, r10, r11, r12, r13,
                msg1_o, efT_o, yT_o):
    pp = pos8[...]
    st = shT[...]
    vx = pp[0:1] - pp[3:4] + st[0:1]
    vy = pp[1:2] - pp[4:5] + st[1:2]
    vz = pp[2:3] - pp[5:6] + st[2:3]
    r = jnp.sqrt(vx * vx + vy * vy + vz * vz)
    rinv = 1.0 / (r + 1e-9)
    s3 = 3.0 ** 0.5
    yT = jnp.concatenate([jnp.ones_like(r), s3 * vx * rinv,
                          s3 * vy * rinv, s3 * vz * rinv], axis=0)
    yT_o[...] = yT
    k = (lax.broadcasted_iota(jnp.int32, (NB, 1), 0) + 1).astype(jnp.float32)
    bes = jnp.sqrt(2.0 / RMAX) * jnp.sin(k * (jnp.pi / RMAX) * r) * rinv
    x = r / RMAX
    x2 = x * x
    x4 = x2 * x2
    x5 = x4 * x
    x6 = x5 * x
    x7 = x6 * x
    p = 5.0
    fcut = (1.0 - ((p + 1) * (p + 2) / 2.0) * x5 + p * (p + 2) * x6
            - (p * (p + 1) / 2.0) * x7)
    fcut = jnp.where(x < 1.0, fcut, 0.0)
    ef = bes * fcut
    efT_o[...] = ef

    r1 = _radialT(ef, r10, r11, r12, r13)
    y = _cols4(yT)
    hs = hup1s[...]
    rows = []
    for m in range(SH):
        rl = r1[:, :C] if m == 0 else r1[:, C:]
        rows.append((rl * y[:, m:m + 1] * hs)[None])
    msg1_o[...] = jnp.concatenate(rows, axis=0)


def _radialT(efT, w0, w1, w2, w3):
    h = _silu(lax.dot_general(efT, w0[...], (((0,), (0,)), ((), ())),
                              preferred_element_type=jnp.float32))
    h = _silu(jnp.dot(h, w1[...], preferred_element_type=jnp.float32))
    h = _silu(jnp.dot(h, w2[...], preferred_element_type=jnp.float32))
    return jnp.dot(h, w3[...], preferred_element_type=jnp.float32)


def _cols4(yT):
    i0 = lax.broadcasted_iota(jnp.int32, (4, 4), 0)
    i1 = lax.broadcasted_iota(jnp.int32, (4, 4), 1)
    eye4 = jnp.where(i0 == i1, 1.0, 0.0).astype(jnp.float32)
    return lax.dot_general(yT, eye4, (((0,), (0,)), ((), ())),
                           preferred_element_type=jnp.float32)


def _edge2_body(efT_i, yT_i, hup2s, r20, r21, r22, r23, msg2_o):
    r2 = _radialT(efT_i[...], r20, r21, r22, r23)
    y = _cols4(yT_i[...])
    hs = hup2s[...]
    rows = []
    for m in range(SH):
        rl = r2[:, :C] if m == 0 else r2[:, C:]
        rows.append((rl * y[:, m:m + 1] * hs)[None])
    msg2_o[...] = jnp.concatenate(rows, axis=0)


def _node_mid_body(agg1, wmix1, na, h0, pc1, wsc1, sc1a, wup2, wr1,
                   hup2_o, ne1_o):
    inv = 1.0 / AVG
    parts = []
    for m in range(SH):
        wl = wmix1[...][0] if m == 0 else wmix1[...][1]
        parts.append(jnp.dot(agg1[...][m] * inv, wl,
                             preferred_element_type=jnp.float32))
    m1 = jnp.concatenate(parts, axis=1)
    na_v = na[...]
    c0 = jnp.dot(na_v, pc1[...][0], preferred_element_type=jnp.float32)
    c1 = jnp.dot(na_v, pc1[...][1], preferred_element_type=jnp.float32)
    c2 = jnp.dot(na_v, pc1[...][2], preferred_element_type=jnp.float32)
    sc1 = (jnp.dot(h0[...], wsc1[...], preferred_element_type=jnp.float32)
           * jnp.dot(na_v, sc1a[...], preferred_element_type=jnp.float32))
    h1 = c0 * m1 + c1 * m1 * m1 + c2 * m1 * m1 * m1 + sc1
    hs = h1[:, :C]
    ne1_o[...] = jnp.dot(hs, wr1[...], preferred_element_type=jnp.float32)
    hup2_o[...] = jnp.dot(hs, wup2[...], preferred_element_type=jnp.float32)


def _node_out_body(agg2, wmix2, na, hup2, pc2, wout2, sc2a, nlw1, nlw2, ae,
                   ne1, nrg_o):
    inv = 1.0 / AVG
    parts = []
    for m in range(SH):
        wl = wmix2[...][0] if m == 0 else wmix2[...][1]
        parts.append(jnp.dot(agg2[...][m] * inv, wl,
                             preferred_element_type=jnp.float32))
    m2 = jnp.concatenate(parts, axis=1)
    na_v = na[...]
    c0 = jnp.dot(na_v, pc2[...][0], preferred_element_type=jnp.float32)
    c1 = jnp.dot(na_v, pc2[...][1], preferred_element_type=jnp.float32)
    c2 = jnp.dot(na_v, pc2[...][2], preferred_element_type=jnp.float32)
    p = c0 * m2 + c1 * m2 * m2 + c2 * m2 * m2 * m2
    h2 = (jnp.dot(p, wout2[...], preferred_element_type=jnp.float32)
          + hup2[...] * jnp.dot(na_v, sc2a[...],
                                preferred_element_type=jnp.float32))
    t = _silu(jnp.dot(h2, nlw1[...], preferred_element_type=jnp.float32))
    ne2 = jnp.dot(t, nlw2[...], preferred_element_type=jnp.float32)
    e0 = jnp.dot(na_v, ae[...], preferred_element_type=jnp.float32)
    nrg_o[...] = e0 + ne1[...] + ne2


def _graph_sum_body(ne, out):
    out[...] = jnp.sum(ne[...], axis=1)[None, :]


GB0 = 16
GB1 = 64


def _sc_gather3(px, py, pz, hup1, src2d, dst2d, pos8_o, hup1s_o,
                px_v, py_v, pz_v, idxs_all, idxd_all, rh0, rh1, pb0, pb1,
                sg0, sg1, sw0, sw1, sp0, sp1):
    c = lax.axis_index("c")
    s = lax.axis_index("s")
    pltpu.sync_copy(px, px_v)
    pltpu.sync_copy(py, py_v)
    pltpu.sync_copy(pz, pz_v)
    rh = (rh0, rh1)
    pb = (pb0, pb1)
    sg = (sg0, sg1)
    sw = (sw0, sw1)
    sp = (sp0, sp1)

    def run(row0, nb):
        pltpu.sync_copy(src2d.at[pl.ds(row0, nb)], idxs_all.at[pl.ds(0, nb)])
        pltpu.sync_copy(dst2d.at[pl.ds(row0, nb)], idxd_all.at[pl.ds(0, nb)])

        def fire_g(j, b):
            pltpu.async_copy(hup1.at[idxs_all.at[j]], rh[b], sg[b])

        def wait_g(b):
            pltpu.make_async_copy(hup1.at[idxs_all.at[0]], rh[b],
                                  sg[b]).wait()

        def pos_gather(j, b):
            def chunk(k, carry):
                sl = pl.ds(k * 16, 16)
                iv_s = idxs_all[j, sl]
                iv_d = idxd_all[j, sl]
                pb[b][0, sl] = plsc.load_gather(px_v, [iv_s])
                pb[b][1, sl] = plsc.load_gather(py_v, [iv_s])
                pb[b][2, sl] = plsc.load_gather(pz_v, [iv_s])
                pb[b][3, sl] = plsc.load_gather(px_v, [iv_d])
                pb[b][4, sl] = plsc.load_gather(py_v, [iv_d])
                pb[b][5, sl] = plsc.load_gather(pz_v, [iv_d])
                return carry

            lax.fori_loop(0, IB // 16, chunk, 0)

        def fire_w(j, b):
            base = (row0 + j) * IB
            pltpu.async_copy(rh[b], hup1s_o.at[pl.ds(base, IB)], sw[b])
            pltpu.async_copy(pb[b], pos8_o.at[:, pl.ds(base, IB)], sp[b])

        def wait_sw(b):
            pltpu.make_async_copy(rh[b], hup1s_o.at[pl.ds(0, IB)],
                                  sw[b]).wait()

        def wait_sp(b):
            pltpu.make_async_copy(pb[b], pos8_o.at[:, pl.ds(0, IB)],
                                  sp[b]).wait()

        fire_g(0, 0)

        def pair(gp, carry):
            for b in (0, 1):
                j = 2 * gp + b
                ob = 1 - b
                pl.when(j >= 2)(lambda: wait_sp(b))
                pos_gather(j, b)
                wait_g(b)
                pl.when(j >= 1)(lambda: wait_sw(ob))
                pl.when(j + 1 < nb)(lambda: fire_g(j + 1, ob))
                fire_w(j, b)
            return carry

        lax.fori_loop(0, nb // 2, pair, 0)
        wait_sw(1)
        wait_sp(0)
        wait_sp(1)

    pl.when(c == 1)(lambda: run(s * GB0, GB0))
    pl.when(c == 0)(lambda: run(NS * GB0 + s * GB1, GB1))


def _sc_gather1(hup2, src2d, hup2s_o, idxs_all, rh0, rh1, sg0, sg1,
                sw0, sw1):
    c = lax.axis_index("c")
    s = lax.axis_index("s")
    rh = (rh0, rh1)
    sg = (sg0, sg1)
    sw = (sw0, sw1)

    def run(row0, nb):
        pltpu.sync_copy(src2d.at[pl.ds(row0, nb)], idxs_all.at[pl.ds(0, nb)])

        def fire_g(j, b):
            pltpu.async_copy(hup2.at[idxs_all.at[j]], rh[b], sg[b])

        def wait_g(b):
            pltpu.make_async_copy(hup2.at[idxs_all.at[0]], rh[b],
                                  sg[b]).wait()

        def fire_w(j, b):
            pltpu.async_copy(rh[b], hup2s_o.at[pl.ds((row0 + j) * IB, IB)],
                             sw[b])

        def wait_w(b):
            pltpu.make_async_copy(rh[b], hup2s_o.at[pl.ds(0, IB)],
                                  sw[b]).wait()

        fire_g(0, 0)

        def pair(gp, carry):
            for b in (0, 1):
                j = 2 * gp + b
                ob = 1 - b
                wait_g(b)
                fire_w(j, b)
                pl.when(j >= 1)(lambda: wait_w(ob))
                pl.when(j + 1 < nb)(lambda: fire_g(j + 1, ob))
            return carry

        lax.fori_loop(0, nb // 2, pair, 0)
        wait_w(1)

    pl.when(c == 1)(lambda: run(s * GB0, GB0))
    pl.when(c == 0)(lambda: run(NS * GB0 + s * GB1, GB1))


def _sc_scatter(msg, dst2d, zeros, agg_o, agg_sp, idx_all, mr0, mr1,
                sm0, sm1, ss0, ss1):
    c = lax.axis_index("c")
    s = lax.axis_index("s")
    row0 = s * TROW
    pltpu.sync_copy(dst2d.at[pl.ds(s * SB, SB)], idx_all)
    mr = (mr0, mr1)
    sm = (sm0, sm1)
    ss = (ss0, ss1)
    for jc in range(2):
        m = 2 * c + jc

        def fire_msg(j, b):
            pltpu.async_copy(msg.at[m, pl.ds((s * SB + j) * IB, IB)],
                             mr[b], sm[b])

        def wait_msg(b):
            pltpu.make_async_copy(msg.at[m, pl.ds(0, IB)], mr[b],
                                  sm[b]).wait()

        def fire_sc(j, b):
            pltpu.async_copy(mr[b], agg_sp.at[idx_all.at[j]], ss[b],
                             add=True)

        def wait_sc(b):
            pltpu.make_async_copy(mr[b], agg_sp.at[idx_all.at[0]],
                                  ss[b]).wait()

        pltpu.sync_copy(zeros, agg_sp.at[pl.ds(row0, NROWS)])
        plsc.subcore_barrier()
        fire_msg(0, 0)

        def pair(gp, carry):
            for b in (0, 1):
                j = 2 * gp + b
                ob = 1 - b
                wait_msg(b)
                fire_sc(j, b)
                pl.when(j >= 1)(lambda: wait_sc(ob))
                pl.when(j + 1 < SB)(lambda: fire_msg(j + 1, ob))
            return carry

        lax.fori_loop(0, SB // 2, pair, 0)
        wait_sc(1)
        plsc.subcore_barrier()
        pltpu.sync_copy(agg_sp.at[pl.ds(row0, NROWS)],
                        agg_o.at[m, pl.ds(row0, NROWS)])


@functools.cache
def _sc_kernels():
    mesh = plsc.VectorSubcoreMesh(core_axis_name="c", subcore_axis_name="s",
                                  num_cores=NC, num_subcores=NS)
    gather3 = pl.kernel(
        _sc_gather3,
        out_type=[jax.ShapeDtypeStruct((8, E_PAD), jnp.float32),
                  jax.ShapeDtypeStruct((E_PAD, C), jnp.float32)],
        mesh=mesh,
        compiler_params=pltpu.CompilerParams(needs_layout_passes=False),
        scratch_types=[pltpu.VMEM((N,), jnp.float32),
                       pltpu.VMEM((N,), jnp.float32),
                       pltpu.VMEM((N,), jnp.float32),
                       pltpu.VMEM((GB1, IB), jnp.int32),
                       pltpu.VMEM((GB1, IB), jnp.int32),
                       pltpu.VMEM((IB, C), jnp.float32),
                       pltpu.VMEM((IB, C), jnp.float32),
                       pltpu.VMEM((8, IB), jnp.float32),
                       pltpu.VMEM((8, IB), jnp.float32),
                       pltpu.SemaphoreType.DMA,
                       pltpu.SemaphoreType.DMA,
                       pltpu.SemaphoreType.DMA,
                       pltpu.SemaphoreType.DMA,
                       pltpu.SemaphoreType.DMA,
                       pltpu.SemaphoreType.DMA],
    )
    gather1 = pl.kernel(
        _sc_gather1,
        out_type=[jax.ShapeDtypeStruct((E_PAD, C), jnp.float32)],
        mesh=mesh,
        scratch_types=[pltpu.VMEM((GB1, IB), jnp.int32),
                       pltpu.VMEM((IB, C), jnp.float32),
                       pltpu.VMEM((IB, C), jnp.float32),
                       pltpu.SemaphoreType.DMA,
                       pltpu.SemaphoreType.DMA,
                       pltpu.SemaphoreType.DMA,
                       pltpu.SemaphoreType.DMA],
    )
    scatter = pl.kernel(
        _sc_scatter,
        out_type=[jax.ShapeDtypeStruct((SH, N, C), jnp.float32)],
        mesh=mesh,
        scratch_types=[pltpu.VMEM_SHARED((N, C), jnp.float32),
                       pltpu.VMEM((SB, IB), jnp.int32),
                       pltpu.VMEM((IB, C), jnp.float32),
                       pltpu.VMEM((IB, C), jnp.float32),
                       pltpu.SemaphoreType.DMA,
                       pltpu.SemaphoreType.DMA,
                       pltpu.SemaphoreType.DMA,
                       pltpu.SemaphoreType.DMA],
    )
    return gather3, gather1, scatter


def _full(shape):
    return pl.BlockSpec(shape, lambda *_: tuple(0 for _ in shape))


def _node_pre(na, we, wup1):
    return pl.pallas_call(
        _node_pre_body,
        out_shape=[jax.ShapeDtypeStruct((N, C), jnp.float32),
                   jax.ShapeDtypeStruct((N, C), jnp.float32)],
    )(na, we, wup1)


def _edge1(pos8, shT, hup1s, rad1):
    eb = lambda w: pl.BlockSpec((BE, w), lambda i: (i, 0))
    return pl.pallas_call(
        _edge1_body,
        grid=(EGRID,),
        in_specs=[pl.BlockSpec((8, BE), lambda i: (0, i)),
                  pl.BlockSpec((4, BE), lambda i: (0, i)), eb(C),
                  _full((NB, 64)), _full((64, 64)), _full((64, 64)),
                  _full((64, 2 * C))],
        out_specs=[pl.BlockSpec((SH, BE, C), lambda i: (0, i, 0)),
                   pl.BlockSpec((NB, BE), lambda i: (0, i)),
                   pl.BlockSpec((4, BE), lambda i: (0, i))],
        out_shape=[jax.ShapeDtypeStruct((SH, E_PAD, C), jnp.float32),
                   jax.ShapeDtypeStruct((NB, E_PAD), jnp.float32),
                   jax.ShapeDtypeStruct((4, E_PAD), jnp.float32)],
    )(pos8, shT, hup1s, *rad1)


def _edge2(efT, yT, hup2s, rad2):
    eb = lambda w: pl.BlockSpec((BE, w), lambda i: (i, 0))
    return pl.pallas_call(
        _edge2_body,
        grid=(EGRID,),
        in_specs=[pl.BlockSpec((NB, BE), lambda i: (0, i)),
                  pl.BlockSpec((4, BE), lambda i: (0, i)), eb(C),
                  _full((NB, 64)), _full((64, 64)), _full((64, 64)),
                  _full((64, 2 * C))],
        out_specs=[pl.BlockSpec((SH, BE, C), lambda i: (0, i, 0))],
        out_shape=[jax.ShapeDtypeStruct((SH, E_PAD, C), jnp.float32)],
    )(efT, yT, hup2s, *rad2)


def _node_mid(agg1, wmix1, na, h0, pc1, wsc1, sc1a, wup2, wr1):
    nb = lambda w: pl.BlockSpec((BN, w), lambda i: (i, 0))
    return pl.pallas_call(
        _node_mid_body,
        grid=(NGRID,),
        in_specs=[pl.BlockSpec((SH, BN, C), lambda i: (0, i, 0)),
                  _full((2, C, C)), nb(NE), nb(C), _full((3, NE, D)),
                  _full((C, D)), _full((NE, 1)), _full((C, C)),
                  _full((C, 1))],
        out_specs=[nb(C), nb(1)],
        out_shape=[jax.ShapeDtypeStruct((N, C), jnp.float32),
                   jax.ShapeDtypeStruct((N, 1), jnp.float32)],
    )(agg1, wmix1, na, h0, pc1, wsc1, sc1a, wup2, wr1)


def _node_out(agg2, wmix2, na, hup2, pc2, wout2, sc2a, nlw1, nlw2, ae, ne1):
    nb = lambda w: pl.BlockSpec((BN, w), lambda i: (i, 0))
    return pl.pallas_call(
        _node_out_body,
        grid=(NGRID,),
        in_specs=[pl.BlockSpec((SH, BN, C), lambda i: (0, i, 0)),
                  _full((2, C, C)), nb(NE), nb(C), _full((3, NE, D)),
                  _full((D, C)), _full((NE, 1)), _full((C, 16)),
                  _full((16, 1)), _full((NE, 1)), nb(1)],
        out_specs=[nb(1)],
        out_shape=[jax.ShapeDtypeStruct((N, 1), jnp.float32)],
    )(agg2, wmix2, na, hup2, pc2, wout2, sc2a, nlw1, nlw2, ae, ne1)


def _graph_sum(ne_rows):
    return pl.pallas_call(
        _graph_sum_body,
        out_shape=jax.ShapeDtypeStruct((1, G), jnp.float32),
    )(ne_rows)


def kernel(positions, node_attrs, shifts, params, edge_index, batch):
    p = params
    src2d = jnp.pad(edge_index[0].astype(jnp.int32),
                    (0, E_PAD - E)).reshape(E_PAD // IB, IB)
    dst2d = jnp.pad(edge_index[1].astype(jnp.int32),
                    (0, E_PAD - E)).reshape(E_PAD // IB, IB)
    shT = jnp.pad(shifts.T, ((0, 1), (0, E_PAD - E)))
    px = jnp.asarray(positions[:, 0])
    py = jnp.asarray(positions[:, 1])
    pz = jnp.asarray(positions[:, 2])
    zeros = jnp.zeros((NROWS, C), jnp.float32)

    gather3, gather1, scatter = _sc_kernels()
    h0, hup1 = _node_pre(node_attrs, p['W_embed'], p['W_up1'])
    pos8, hup1s = gather3(px, py, pz, hup1, src2d, dst2d)
    msg1, efT, yT = _edge1(pos8, shT, hup1s, p['rad1'])
    agg1, = scatter(msg1, dst2d, zeros)
    hup2, ne1 = _node_mid(agg1, p['W_mix1'], node_attrs, h0, p['prod1_c'],
                          p['W_sc1'], p['sc1_a'][:, None], p['W_up2'],
                          p['w_r1'])
    hup2s, = gather1(hup2, src2d)
    msg2, = _edge2(efT, yT, hup2s, p['rad2'])
    agg2, = scatter(msg2, dst2d, zeros)
    ne, = _node_out(agg2, p['W_mix2'], node_attrs, hup2, p['prod2_c'],
                   p['W_out2'], p['sc2_a'][:, None], p['nl_w1'], p['nl_w2'],
                   p['atomic_energies'][:, None], ne1)
    return _graph_sum(ne.reshape(G, N // G)).reshape(G)


# speedup vs baseline: 1.2553x; 1.2352x over previous
import functools

import jax
import jax.numpy as jnp
from jax import lax
from jax.experimental import pallas as pl
from jax.experimental.pallas import tpu as pltpu
from jax.experimental.pallas import tpu_sc as plsc

N = 10000
E = 160000
NE = 10
C = 128
SH = 4
NB = 8
RMAX = 5.0
G = 4
AVG = 16.0
D = SH * C

NC = 2
NS = 16
NW = NC * NS
IB = 128
GB = 40
E_PAD = NW * GB * IB
SB = E_PAD // NS // IB
TROW = 624
NROWS = 640

BE = 2048
EGRID = E_PAD // BE
BN = 2000
NGRID = N // BN


def _silu(x):
    return x * jax.nn.sigmoid(x)


def _node_pre_body(na, we, wup1, h0_o, hup1_o):
    h0 = jnp.dot(na[...], we[...], preferred_element_type=jnp.float32)
    h0_o[...] = h0
    hup1_o[...] = jnp.dot(h0, wup1[...], preferred_element_type=jnp.float32)


def _edge1_body(pos8, shT, hup1s, r10, r11, r12, r13,
                msg1_o, efT_o, yT_o):
    pp = pos8[...]
    st = shT[...]
    vx = pp[0:1] - pp[3:4] + st[0:1]
    vy = pp[1:2] - pp[4:5] + st[1:2]
    vz = pp[2:3] - pp[5:6] + st[2:3]
    r = jnp.sqrt(vx * vx + vy * vy + vz * vz)
    rinv = 1.0 / (r + 1e-9)
    s3 = 3.0 ** 0.5
    yT = jnp.concatenate([jnp.ones_like(r), s3 * vx * rinv,
                          s3 * vy * rinv, s3 * vz * rinv], axis=0)
    yT_o[...] = yT
    k = (lax.broadcasted_iota(jnp.int32, (NB, 1), 0) + 1).astype(jnp.float32)
    bes = jnp.sqrt(2.0 / RMAX) * jnp.sin(k * (jnp.pi / RMAX) * r) * rinv
    x = r / RMAX
    x2 = x * x
    x4 = x2 * x2
    x5 = x4 * x
    x6 = x5 * x
    x7 = x6 * x
    p = 5.0
    fcut = (1.0 - ((p + 1) * (p + 2) / 2.0) * x5 + p * (p + 2) * x6
            - (p * (p + 1) / 2.0) * x7)
    fcut = jnp.where(x < 1.0, fcut, 0.0)
    ef = bes * fcut
    efT_o[...] = ef

    r1 = _radialT(ef, r10, r11, r12, r13)
    y = _cols4(yT)
    hs = hup1s[...]
    rows = []
    for m in range(SH):
        rl = r1[:, :C] if m == 0 else r1[:, C:]
        rows.append((rl * y[:, m:m + 1] * hs)[None])
    msg1_o[...] = jnp.concatenate(rows, axis=0)


def _radialT(efT, w0, w1, w2, w3):
    h = _silu(lax.dot_general(efT, w0[...], (((0,), (0,)), ((), ())),
                              preferred_element_type=jnp.float32))
    h = _silu(jnp.dot(h, w1[...], preferred_element_type=jnp.float32))
    h = _silu(jnp.dot(h, w2[...], preferred_element_type=jnp.float32))
    return jnp.dot(h, w3[...], preferred_element_type=jnp.float32)


def _cols4(yT):
    i0 = lax.broadcasted_iota(jnp.int32, (4, 4), 0)
    i1 = lax.broadcasted_iota(jnp.int32, (4, 4), 1)
    eye4 = jnp.where(i0 == i1, 1.0, 0.0).astype(jnp.float32)
    return lax.dot_general(yT, eye4, (((0,), (0,)), ((), ())),
                           preferred_element_type=jnp.float32)


def _edge2_body(efT_i, yT_i, hup2s, r20, r21, r22, r23, msg2_o):
    r2 = _radialT(efT_i[...], r20, r21, r22, r23)
    y = _cols4(yT_i[...])
    hs = hup2s[...]
    rows = []
    for m in range(SH):
        rl = r2[:, :C] if m == 0 else r2[:, C:]
        rows.append((rl * y[:, m:m + 1] * hs)[None])
    msg2_o[...] = jnp.concatenate(rows, axis=0)


def _node_mid_body(agg1, wmix1, na, h0, pc1, wsc1, sc1a, wup2, wr1,
                   hup2_o, ne1_o):
    inv = 1.0 / AVG
    parts = []
    for m in range(SH):
        wl = wmix1[...][0] if m == 0 else wmix1[...][1]
        parts.append(jnp.dot(agg1[...][m] * inv, wl,
                             preferred_element_type=jnp.float32))
    m1 = jnp.concatenate(parts, axis=1)
    na_v = na[...]
    c0 = jnp.dot(na_v, pc1[...][0], preferred_element_type=jnp.float32)
    c1 = jnp.dot(na_v, pc1[...][1], preferred_element_type=jnp.float32)
    c2 = jnp.dot(na_v, pc1[...][2], preferred_element_type=jnp.float32)
    sc1 = (jnp.dot(h0[...], wsc1[...], preferred_element_type=jnp.float32)
           * jnp.dot(na_v, sc1a[...], preferred_element_type=jnp.float32))
    h1 = c0 * m1 + c1 * m1 * m1 + c2 * m1 * m1 * m1 + sc1
    hs = h1[:, :C]
    ne1_o[...] = jnp.dot(hs, wr1[...], preferred_element_type=jnp.float32)
    hup2_o[...] = jnp.dot(hs, wup2[...], preferred_element_type=jnp.float32)


def _node_out_body(agg2, wmix2, na, hup2, pc2, wout2, sc2a, nlw1, nlw2, ae,
                   ne1, nrg_o):
    inv = 1.0 / AVG
    parts = []
    for m in range(SH):
        wl = wmix2[...][0] if m == 0 else wmix2[...][1]
        parts.append(jnp.dot(agg2[...][m] * inv, wl,
                             preferred_element_type=jnp.float32))
    m2 = jnp.concatenate(parts, axis=1)
    na_v = na[...]
    c0 = jnp.dot(na_v, pc2[...][0], preferred_element_type=jnp.float32)
    c1 = jnp.dot(na_v, pc2[...][1], preferred_element_type=jnp.float32)
    c2 = jnp.dot(na_v, pc2[...][2], preferred_element_type=jnp.float32)
    p = c0 * m2 + c1 * m2 * m2 + c2 * m2 * m2 * m2
    h2 = (jnp.dot(p, wout2[...], preferred_element_type=jnp.float32)
          + hup2[...] * jnp.dot(na_v, sc2a[...],
                                preferred_element_type=jnp.float32))
    t = _silu(jnp.dot(h2, nlw1[...], preferred_element_type=jnp.float32))
    ne2 = jnp.dot(t, nlw2[...], preferred_element_type=jnp.float32)
    e0 = jnp.dot(na_v, ae[...], preferred_element_type=jnp.float32)
    nrg_o[...] = e0 + ne1[...] + ne2


def _graph_sum_body(ne, out):
    out[...] = jnp.sum(ne[...], axis=1)[None, :]


GB0 = 40
GB1 = 40


def _sc_gather3(px, py, pz, hup1, src2d, dst2d, pos8_o, hup1s_o,
                px_v, py_v, pz_v, idxs_all, idxd_all, rh0, rh1,
                pb0, pb1, sg0, sg1, sw0, sw1, sp0, sp1):
    c = lax.axis_index("c")
    s = lax.axis_index("s")
    pltpu.sync_copy(px, px_v)
    pltpu.sync_copy(py, py_v)
    pltpu.sync_copy(pz, pz_v)
    rh = (rh0, rh1)
    pb = (pb0, pb1)
    sg = (sg0, sg1)
    sw = (sw0, sw1)
    sp = (sp0, sp1)

    def run(row0, nb):
        pltpu.sync_copy(src2d.at[pl.ds(row0, nb)], idxs_all.at[pl.ds(0, nb)])
        pltpu.sync_copy(dst2d.at[pl.ds(row0, nb)], idxd_all.at[pl.ds(0, nb)])

        def fire_g(j, b):
            pltpu.async_copy(hup1.at[idxs_all.at[j]], rh[b], sg[b])

        def wait_g(b):
            pltpu.make_async_copy(hup1.at[idxs_all.at[0]], rh[b],
                                  sg[b]).wait()

        def pos_gather(j, b):
            def chunk(k, carry):
                sl = pl.ds(k * 16, 16)
                iv_s = idxs_all[j, sl]
                iv_d = idxd_all[j, sl]
                pb[b][0, sl] = plsc.load_gather(px_v, [iv_s])
                pb[b][1, sl] = plsc.load_gather(py_v, [iv_s])
                pb[b][2, sl] = plsc.load_gather(pz_v, [iv_s])
                pb[b][3, sl] = plsc.load_gather(px_v, [iv_d])
                pb[b][4, sl] = plsc.load_gather(py_v, [iv_d])
                pb[b][5, sl] = plsc.load_gather(pz_v, [iv_d])
                return carry

            lax.fori_loop(0, IB // 16, chunk, 0)

        def fire_w(j, b):
            base = (row0 + j) * IB
            pltpu.async_copy(rh[b], hup1s_o.at[pl.ds(base, IB)], sw[b])
            pltpu.async_copy(pb[b], pos8_o.at[:, pl.ds(base, IB)], sp[b])

        def wait_sw(b):
            pltpu.make_async_copy(rh[b], hup1s_o.at[pl.ds(0, IB)],
                                  sw[b]).wait()

        def wait_sp(b):
            pltpu.make_async_copy(pb[b], pos8_o.at[:, pl.ds(0, IB)],
                                  sp[b]).wait()

        fire_g(0, 0)

        def pair(gp, carry):
            for b in (0, 1):
                j = 2 * gp + b
                ob = 1 - b
                pl.when(j >= 2)(lambda: wait_sp(b))
                pos_gather(j, b)
                wait_g(b)
                pl.when(j >= 1)(lambda: wait_sw(ob))
                pl.when(j + 1 < nb)(lambda: fire_g(j + 1, ob))
                fire_w(j, b)
            return carry

        lax.fori_loop(0, nb // 2, pair, 0)
        wait_sw(1)
        wait_sp(0)
        wait_sp(1)

    pl.when(c == 1)(lambda: run(s * GB0, GB0))
    pl.when(c == 0)(lambda: run(NS * GB0 + s * GB1, GB1))


def _sc_gather1(hup2, src2d, hup2s_o, hup_sp, idxs_all, rh0, rh1, sg0, sg1,
                sw0, sw1):
    c = lax.axis_index("c")
    s = lax.axis_index("s")
    pltpu.sync_copy(hup2.at[pl.ds(s * TROW, NROWS)],
                    hup_sp.at[pl.ds(s * TROW, NROWS)])
    plsc.subcore_barrier()
    rh = (rh0, rh1)
    sg = (sg0, sg1)
    sw = (sw0, sw1)

    def run(row0, nb):
        pltpu.sync_copy(src2d.at[pl.ds(row0, nb)], idxs_all.at[pl.ds(0, nb)])

        def fire_g(j, b):
            pltpu.async_copy(hup_sp.at[idxs_all.at[j]], rh[b], sg[b])

        def wait_g(b):
            pltpu.make_async_copy(hup_sp.at[idxs_all.at[0]], rh[b],
                                  sg[b]).wait()

        def fire_w(j, b):
            pltpu.async_copy(rh[b], hup2s_o.at[pl.ds((row0 + j) * IB, IB)],
                             sw[b])

        def wait_w(b):
            pltpu.make_async_copy(rh[b], hup2s_o.at[pl.ds(0, IB)],
                                  sw[b]).wait()

        fire_g(0, 0)

        def pair(gp, carry):
            for b in (0, 1):
                j = 2 * gp + b
                ob = 1 - b
                wait_g(b)
                fire_w(j, b)
                pl.when(j >= 1)(lambda: wait_w(ob))
                pl.when(j + 1 < nb)(lambda: fire_g(j + 1, ob))
            return carry

        lax.fori_loop(0, nb // 2, pair, 0)
        wait_w(1)

    pl.when(c == 1)(lambda: run(s * GB0, GB0))
    pl.when(c == 0)(lambda: run(NS * GB0 + s * GB1, GB1))


def _sc_scatter(msg, dst2d, zeros, agg_o, agg_sp, idx_all, mr0, mr1,
                sm0, sm1, ss0, ss1):
    c = lax.axis_index("c")
    s = lax.axis_index("s")
    row0 = s * TROW
    pltpu.sync_copy(dst2d.at[pl.ds(s * SB, SB)], idx_all)
    mr = (mr0, mr1)
    sm = (sm0, sm1)
    ss = (ss0, ss1)
    for jc in range(2):
        m = 2 * c + jc

        def fire_msg(j, b):
            pltpu.async_copy(msg.at[m, pl.ds((s * SB + j) * IB, IB)],
                             mr[b], sm[b])

        def wait_msg(b):
            pltpu.make_async_copy(msg.at[m, pl.ds(0, IB)], mr[b],
                                  sm[b]).wait()

        def fire_sc(j, b):
            pltpu.async_copy(mr[b], agg_sp.at[idx_all.at[j]], ss[b],
                             add=True)

        def wait_sc(b):
            pltpu.make_async_copy(mr[b], agg_sp.at[idx_all.at[0]],
                                  ss[b]).wait()

        pltpu.sync_copy(zeros, agg_sp.at[pl.ds(row0, NROWS)])
        plsc.subcore_barrier()
        fire_msg(0, 0)

        def pair(gp, carry):
            for b in (0, 1):
                j = 2 * gp + b
                ob = 1 - b
                wait_msg(b)
                fire_sc(j, b)
                pl.when(j >= 1)(lambda: wait_sc(ob))
                pl.when(j + 1 < SB)(lambda: fire_msg(j + 1, ob))
            return carry

        lax.fori_loop(0, SB // 2, pair, 0)
        wait_sc(1)
        plsc.subcore_barrier()
        pltpu.sync_copy(agg_sp.at[pl.ds(row0, NROWS)],
                        agg_o.at[m, pl.ds(row0, NROWS)])


@functools.cache
def _sc_kernels():
    mesh = plsc.VectorSubcoreMesh(core_axis_name="c", subcore_axis_name="s",
                                  num_cores=NC, num_subcores=NS)
    gather3 = pl.kernel(
        _sc_gather3,
        out_type=[jax.ShapeDtypeStruct((8, E_PAD), jnp.float32),
                  jax.ShapeDtypeStruct((E_PAD, C), jnp.float32)],
        mesh=mesh,
        compiler_params=pltpu.CompilerParams(needs_layout_passes=False),
        scratch_types=[pltpu.VMEM((N,), jnp.float32),
                       pltpu.VMEM((N,), jnp.float32),
                       pltpu.VMEM((N,), jnp.float32),
                       pltpu.VMEM((GB1, IB), jnp.int32),
                       pltpu.VMEM((GB1, IB), jnp.int32),
                       pltpu.VMEM((IB, C), jnp.float32),
                       pltpu.VMEM((IB, C), jnp.float32),
                       pltpu.VMEM((8, IB), jnp.float32),
                       pltpu.VMEM((8, IB), jnp.float32),
                       pltpu.SemaphoreType.DMA,
                       pltpu.SemaphoreType.DMA,
                       pltpu.SemaphoreType.DMA,
                       pltpu.SemaphoreType.DMA,
                       pltpu.SemaphoreType.DMA,
                       pltpu.SemaphoreType.DMA],
    )
    gather1 = pl.kernel(
        _sc_gather1,
        out_type=[jax.ShapeDtypeStruct((E_PAD, C), jnp.float32)],
        mesh=mesh,
        scratch_types=[pltpu.VMEM_SHARED((N, C), jnp.float32),
                       pltpu.VMEM((GB1, IB), jnp.int32),
                       pltpu.VMEM((IB, C), jnp.float32),
                       pltpu.VMEM((IB, C), jnp.float32),
                       pltpu.SemaphoreType.DMA,
                       pltpu.SemaphoreType.DMA,
                       pltpu.SemaphoreType.DMA,
                       pltpu.SemaphoreType.DMA],
    )
    scatter = pl.kernel(
        _sc_scatter,
        out_type=[jax.ShapeDtypeStruct((SH, N, C), jnp.float32)],
        mesh=mesh,
        scratch_types=[pltpu.VMEM_SHARED((N, C), jnp.float32),
                       pltpu.VMEM((SB, IB), jnp.int32),
                       pltpu.VMEM((IB, C), jnp.float32),
                       pltpu.VMEM((IB, C), jnp.float32),
                       pltpu.SemaphoreType.DMA,
                       pltpu.SemaphoreType.DMA,
                       pltpu.SemaphoreType.DMA,
                       pltpu.SemaphoreType.DMA],
    )
    return gather3, gather1, scatter


def _full(shape):
    return pl.BlockSpec(shape, lambda *_: tuple(0 for _ in shape))


def _node_pre(na, we, wup1):
    return pl.pallas_call(
        _node_pre_body,
        out_shape=[jax.ShapeDtypeStruct((N, C), jnp.float32),
                   jax.ShapeDtypeStruct((N, C), jnp.float32)],
    )(na, we, wup1)


def _edge1(pos8, shT, hup1s, rad1):
    eb = lambda w: pl.BlockSpec((BE, w), lambda i: (i, 0))
    return pl.pallas_call(
        _edge1_body,
        grid=(EGRID,),
        in_specs=[pl.BlockSpec((8, BE), lambda i: (0, i)),
                  pl.BlockSpec((4, BE), lambda i: (0, i)), eb(C),
                  _full((NB, 64)), _full((64, 64)), _full((64, 64)),
                  _full((64, 2 * C))],
        out_specs=[pl.BlockSpec((SH, BE, C), lambda i: (0, i, 0)),
                   pl.BlockSpec((NB, BE), lambda i: (0, i)),
                   pl.BlockSpec((4, BE), lambda i: (0, i))],
        out_shape=[jax.ShapeDtypeStruct((SH, E_PAD, C), jnp.float32),
                   jax.ShapeDtypeStruct((NB, E_PAD), jnp.float32),
                   jax.ShapeDtypeStruct((4, E_PAD), jnp.float32)],
    )(pos8, shT, hup1s, *rad1)


def _edge2(efT, yT, hup2s, rad2):
    eb = lambda w: pl.BlockSpec((BE, w), lambda i: (i, 0))
    return pl.pallas_call(
        _edge2_body,
        grid=(EGRID,),
        in_specs=[pl.BlockSpec((NB, BE), lambda i: (0, i)),
                  pl.BlockSpec((4, BE), lambda i: (0, i)), eb(C),
                  _full((NB, 64)), _full((64, 64)), _full((64, 64)),
                  _full((64, 2 * C))],
        out_specs=[pl.BlockSpec((SH, BE, C), lambda i: (0, i, 0))],
        out_shape=[jax.ShapeDtypeStruct((SH, E_PAD, C), jnp.float32)],
    )(efT, yT, hup2s, *rad2)


def _node_mid(agg1, wmix1, na, h0, pc1, wsc1, sc1a, wup2, wr1):
    nb = lambda w: pl.BlockSpec((BN, w), lambda i: (i, 0))
    return pl.pallas_call(
        _node_mid_body,
        grid=(NGRID,),
        in_specs=[pl.BlockSpec((SH, BN, C), lambda i: (0, i, 0)),
                  _full((2, C, C)), nb(NE), nb(C), _full((3, NE, D)),
                  _full((C, D)), _full((NE, 1)), _full((C, C)),
                  _full((C, 1))],
        out_specs=[nb(C), nb(1)],
        out_shape=[jax.ShapeDtypeStruct((N, C), jnp.float32),
                   jax.ShapeDtypeStruct((N, 1), jnp.float32)],
    )(agg1, wmix1, na, h0, pc1, wsc1, sc1a, wup2, wr1)


def _node_out(agg2, wmix2, na, hup2, pc2, wout2, sc2a, nlw1, nlw2, ae, ne1):
    nb = lambda w: pl.BlockSpec((BN, w), lambda i: (i, 0))
    return pl.pallas_call(
        _node_out_body,
        grid=(NGRID,),
        in_specs=[pl.BlockSpec((SH, BN, C), lambda i: (0, i, 0)),
                  _full((2, C, C)), nb(NE), nb(C), _full((3, NE, D)),
                  _full((D, C)), _full((NE, 1)), _full((C, 16)),
                  _full((16, 1)), _full((NE, 1)), nb(1)],
        out_specs=[nb(1)],
        out_shape=[jax.ShapeDtypeStruct((N, 1), jnp.float32)],
    )(agg2, wmix2, na, hup2, pc2, wout2, sc2a, nlw1, nlw2, ae, ne1)


def _graph_sum(ne_rows):
    return pl.pallas_call(
        _graph_sum_body,
        out_shape=jax.ShapeDtypeStruct((1, G), jnp.float32),
    )(ne_rows)


def kernel(positions, node_attrs, shifts, params, edge_index, batch):
    p = params
    src2d = jnp.pad(edge_index[0].astype(jnp.int32),
                    (0, E_PAD - E)).reshape(E_PAD // IB, IB)
    dst2d = jnp.pad(edge_index[1].astype(jnp.int32),
                    (0, E_PAD - E)).reshape(E_PAD // IB, IB)
    shT = jnp.pad(shifts.T, ((0, 1), (0, E_PAD - E)))
    px = jnp.asarray(positions[:, 0])
    py = jnp.asarray(positions[:, 1])
    pz = jnp.asarray(positions[:, 2])
    zeros = jnp.zeros((NROWS, C), jnp.float32)

    gather3, gather1, scatter = _sc_kernels()
    h0, hup1 = _node_pre(node_attrs, p['W_embed'], p['W_up1'])
    pos8, hup1s = gather3(px, py, pz, hup1, src2d, dst2d)
    msg1, efT, yT = _edge1(pos8, shT, hup1s, p['rad1'])
    agg1, = scatter(msg1, dst2d, zeros)
    hup2, ne1 = _node_mid(agg1, p['W_mix1'], node_attrs, h0, p['prod1_c'],
                          p['W_sc1'], p['sc1_a'][:, None], p['W_up2'],
                          p['w_r1'])
    hup2s, = gather1(hup2, src2d)
    msg2, = _edge2(efT, yT, hup2s, p['rad2'])
    agg2, = scatter(msg2, dst2d, zeros)
    ne, = _node_out(agg2, p['W_mix2'], node_attrs, hup2, p['prod2_c'],
                   p['W_out2'], p['sc2_a'][:, None], p['nl_w1'], p['nl_w2'],
                   p['atomic_energies'][:, None], ne1)
    return _graph_sum(ne.reshape(G, N // G)).reshape(G)


# speedup vs baseline: 1.5335x; 1.2216x over previous
import functools

import jax
import jax.numpy as jnp
from jax import lax
from jax.experimental import pallas as pl
from jax.experimental.pallas import tpu as pltpu
from jax.experimental.pallas import tpu_sc as plsc

N = 10000
E = 160000
NE = 10
C = 128
SH = 4
NB = 8
RMAX = 5.0
G = 4
AVG = 16.0
D = SH * C

NC = 2
NS = 16
NW = NC * NS
IB = 128
GB = 40
E_PAD = NW * GB * IB
SB = E_PAD // NS // IB
TROW = 624
NROWS = 640

BE = 2048
EGRID = E_PAD // BE
BN = 2000
NGRID = N // BN


def _silu(x):
    return x * jax.nn.sigmoid(x)


def _node_pre_body(na, we, wup1, h0_o, hup1_o):
    h0 = jnp.dot(na[...], we[...], preferred_element_type=jnp.float32)
    h0_o[...] = h0
    hup1_o[...] = jnp.dot(h0, wup1[...], preferred_element_type=jnp.float32)


def _edge1_body(pos8, shT, hup1s, r10, r11, r12, r13,
                msg1_o, efT_o, yT_o):
    pp = pos8[...]
    st = shT[...]
    vx = pp[0:1] - pp[3:4] + st[0:1]
    vy = pp[1:2] - pp[4:5] + st[1:2]
    vz = pp[2:3] - pp[5:6] + st[2:3]
    r = jnp.sqrt(vx * vx + vy * vy + vz * vz)
    rinv = 1.0 / (r + 1e-9)
    s3 = 3.0 ** 0.5
    yT = jnp.concatenate([jnp.ones_like(r), s3 * vx * rinv,
                          s3 * vy * rinv, s3 * vz * rinv], axis=0)
    yT_o[...] = yT
    k = (lax.broadcasted_iota(jnp.int32, (NB, 1), 0) + 1).astype(jnp.float32)
    bes = jnp.sqrt(2.0 / RMAX) * jnp.sin(k * (jnp.pi / RMAX) * r) * rinv
    x = r / RMAX
    x2 = x * x
    x4 = x2 * x2
    x5 = x4 * x
    x6 = x5 * x
    x7 = x6 * x
    p = 5.0
    fcut = (1.0 - ((p + 1) * (p + 2) / 2.0) * x5 + p * (p + 2) * x6
            - (p * (p + 1) / 2.0) * x7)
    fcut = jnp.where(x < 1.0, fcut, 0.0)
    ef = bes * fcut
    efT_o[...] = ef

    r1 = _radialT(ef, r10, r11, r12, r13)
    y = _cols4(yT)
    hs = hup1s[...]
    rows = []
    for m in range(SH):
        rl = r1[:, :C] if m == 0 else r1[:, C:]
        rows.append((rl * y[:, m:m + 1] * hs)[None])
    msg1_o[...] = jnp.concatenate(rows, axis=0)


def _radialT(efT, w0, w1, w2, w3):
    h = _silu(lax.dot_general(efT, w0[...], (((0,), (0,)), ((), ())),
                              preferred_element_type=jnp.float32))
    h = _silu(jnp.dot(h, w1[...], preferred_element_type=jnp.float32))
    h = _silu(jnp.dot(h, w2[...], preferred_element_type=jnp.float32))
    return jnp.dot(h, w3[...], preferred_element_type=jnp.float32)


def _cols4(yT):
    i0 = lax.broadcasted_iota(jnp.int32, (4, 4), 0)
    i1 = lax.broadcasted_iota(jnp.int32, (4, 4), 1)
    eye4 = jnp.where(i0 == i1, 1.0, 0.0).astype(jnp.float32)
    return lax.dot_general(yT, eye4, (((0,), (0,)), ((), ())),
                           preferred_element_type=jnp.float32)


def _edge2_body(efT_i, yT_i, hup2s, r20, r21, r22, r23, msg2_o):
    r2 = _radialT(efT_i[...], r20, r21, r22, r23)
    y = _cols4(yT_i[...])
    hs = hup2s[...]
    rows = []
    for m in range(SH):
        rl = r2[:, :C] if m == 0 else r2[:, C:]
        rows.append((rl * y[:, m:m + 1] * hs)[None])
    msg2_o[...] = jnp.concatenate(rows, axis=0)


def _node_mid_body(agg1, wmix1, na, h0, pc1, wsc1, sc1a, wup2, wr1,
                   hup2_o, ne1_o):
    inv = 1.0 / AVG
    parts = []
    for m in range(SH):
        wl = wmix1[...][0] if m == 0 else wmix1[...][1]
        parts.append(jnp.dot(agg1[...][m] * inv, wl,
                             preferred_element_type=jnp.float32))
    m1 = jnp.concatenate(parts, axis=1)
    na_v = na[...]
    c0 = jnp.dot(na_v, pc1[...][0], preferred_element_type=jnp.float32)
    c1 = jnp.dot(na_v, pc1[...][1], preferred_element_type=jnp.float32)
    c2 = jnp.dot(na_v, pc1[...][2], preferred_element_type=jnp.float32)
    sc1 = (jnp.dot(h0[...], wsc1[...], preferred_element_type=jnp.float32)
           * jnp.dot(na_v, sc1a[...], preferred_element_type=jnp.float32))
    h1 = c0 * m1 + c1 * m1 * m1 + c2 * m1 * m1 * m1 + sc1
    hs = h1[:, :C]
    ne1_o[...] = jnp.dot(hs, wr1[...], preferred_element_type=jnp.float32)
    hup2_o[...] = jnp.dot(hs, wup2[...], preferred_element_type=jnp.float32)


def _node_out_body(agg2, wmix2, na, hup2, pc2, wout2, sc2a, nlw1, nlw2, ae,
                   ne1, nrg_o):
    inv = 1.0 / AVG
    parts = []
    for m in range(SH):
        wl = wmix2[...][0] if m == 0 else wmix2[...][1]
        parts.append(jnp.dot(agg2[...][m] * inv, wl,
                             preferred_element_type=jnp.float32))
    m2 = jnp.concatenate(parts, axis=1)
    na_v = na[...]
    c0 = jnp.dot(na_v, pc2[...][0], preferred_element_type=jnp.float32)
    c1 = jnp.dot(na_v, pc2[...][1], preferred_element_type=jnp.float32)
    c2 = jnp.dot(na_v, pc2[...][2], preferred_element_type=jnp.float32)
    p = c0 * m2 + c1 * m2 * m2 + c2 * m2 * m2 * m2
    h2 = (jnp.dot(p, wout2[...], preferred_element_type=jnp.float32)
          + hup2[...] * jnp.dot(na_v, sc2a[...],
                                preferred_element_type=jnp.float32))
    t = _silu(jnp.dot(h2, nlw1[...], preferred_element_type=jnp.float32))
    ne2 = jnp.dot(t, nlw2[...], preferred_element_type=jnp.float32)
    e0 = jnp.dot(na_v, ae[...], preferred_element_type=jnp.float32)
    nrg_o[...] = e0 + ne1[...] + ne2


def _graph_sum_body(ne, out):
    out[...] = jnp.sum(ne[...], axis=1)[None, :]


GB0 = 40
GB1 = 40


def _sc_gpos(px, py, pz, src2d, dst2d, pos8_o,
             px_v, py_v, pz_v, idxs_all, idxd_all, pb0, pb1, sp0, sp1):
    c = lax.axis_index("c")
    s = lax.axis_index("s")
    pltpu.sync_copy(px, px_v)
    pltpu.sync_copy(py, py_v)
    pltpu.sync_copy(pz, pz_v)
    pb = (pb0, pb1)
    sp = (sp0, sp1)

    def run(row0, nb):
        pltpu.sync_copy(src2d.at[pl.ds(row0, nb)], idxs_all.at[pl.ds(0, nb)])
        pltpu.sync_copy(dst2d.at[pl.ds(row0, nb)], idxd_all.at[pl.ds(0, nb)])

        def pos_gather(j, b):
            def chunk(k, carry):
                sl = pl.ds(k * 16, 16)
                iv_s = idxs_all[j, sl]
                iv_d = idxd_all[j, sl]
                pb[b][0, sl] = plsc.load_gather(px_v, [iv_s])
                pb[b][1, sl] = plsc.load_gather(py_v, [iv_s])
                pb[b][2, sl] = plsc.load_gather(pz_v, [iv_s])
                pb[b][3, sl] = plsc.load_gather(px_v, [iv_d])
                pb[b][4, sl] = plsc.load_gather(py_v, [iv_d])
                pb[b][5, sl] = plsc.load_gather(pz_v, [iv_d])
                return carry

            lax.fori_loop(0, IB // 16, chunk, 0)

        def fire_w(j, b):
            pltpu.async_copy(pb[b], pos8_o.at[:, pl.ds((row0 + j) * IB, IB)],
                             sp[b])

        def wait_sp(b):
            pltpu.make_async_copy(pb[b], pos8_o.at[:, pl.ds(0, IB)],
                                  sp[b]).wait()

        def pair(gp, carry):
            for b in (0, 1):
                j = 2 * gp + b
                pl.when(j >= 2)(lambda: wait_sp(b))
                pos_gather(j, b)
                fire_w(j, b)
            return carry

        lax.fori_loop(0, nb // 2, pair, 0)
        wait_sp(0)
        wait_sp(1)

    pl.when(c == 1)(lambda: run(s * GB0, GB0))
    pl.when(c == 0)(lambda: run(NS * GB0 + s * GB1, GB1))


def _sc_gather1(hup2, src2d, hup2s_o, hup_sp, idxs_all, rh0, rh1, sg0, sg1,
                sw0, sw1):
    c = lax.axis_index("c")
    s = lax.axis_index("s")
    pltpu.sync_copy(hup2.at[pl.ds(s * TROW, NROWS)],
                    hup_sp.at[pl.ds(s * TROW, NROWS)])
    plsc.subcore_barrier()
    rh = (rh0, rh1)
    sg = (sg0, sg1)
    sw = (sw0, sw1)

    def run(row0, nb):
        pltpu.sync_copy(src2d.at[pl.ds(row0, nb)], idxs_all.at[pl.ds(0, nb)])

        def fire_g(j, b):
            pltpu.async_copy(hup_sp.at[idxs_all.at[j]], rh[b], sg[b])

        def wait_g(b):
            pltpu.make_async_copy(hup_sp.at[idxs_all.at[0]], rh[b],
                                  sg[b]).wait()

        def fire_w(j, b):
            pltpu.async_copy(rh[b], hup2s_o.at[pl.ds((row0 + j) * IB, IB)],
                             sw[b])

        def wait_w(b):
            pltpu.make_async_copy(rh[b], hup2s_o.at[pl.ds(0, IB)],
                                  sw[b]).wait()

        fire_g(0, 0)

        def pair(gp, carry):
            for b in (0, 1):
                j = 2 * gp + b
                ob = 1 - b
                wait_g(b)
                fire_w(j, b)
                pl.when(j >= 1)(lambda: wait_w(ob))
                pl.when(j + 1 < nb)(lambda: fire_g(j + 1, ob))
            return carry

        lax.fori_loop(0, nb // 2, pair, 0)
        wait_w(1)

    pl.when(c == 1)(lambda: run(s * GB0, GB0))
    pl.when(c == 0)(lambda: run(NS * GB0 + s * GB1, GB1))


def _sc_scatter(msg, dst2d, zeros, agg_o, agg_sp, idx_all, mr0, mr1,
                sm0, sm1, ss0, ss1):
    c = lax.axis_index("c")
    s = lax.axis_index("s")
    row0 = s * TROW
    pltpu.sync_copy(dst2d.at[pl.ds(s * SB, SB)], idx_all)
    mr = (mr0, mr1)
    sm = (sm0, sm1)
    ss = (ss0, ss1)
    for jc in range(2):
        m = 2 * c + jc

        def fire_msg(j, b):
            pltpu.async_copy(msg.at[m, pl.ds((s * SB + j) * IB, IB)],
                             mr[b], sm[b])

        def wait_msg(b):
            pltpu.make_async_copy(msg.at[m, pl.ds(0, IB)], mr[b],
                                  sm[b]).wait()

        def fire_sc(j, b):
            pltpu.async_copy(mr[b], agg_sp.at[idx_all.at[j]], ss[b],
                             add=True)

        def wait_sc(b):
            pltpu.make_async_copy(mr[b], agg_sp.at[idx_all.at[0]],
                                  ss[b]).wait()

        pltpu.sync_copy(zeros, agg_sp.at[pl.ds(row0, NROWS)])
        plsc.subcore_barrier()
        fire_msg(0, 0)

        def pair(gp, carry):
            for b in (0, 1):
                j = 2 * gp + b
                ob = 1 - b
                wait_msg(b)
                fire_sc(j, b)
                pl.when(j >= 1)(lambda: wait_sc(ob))
                pl.when(j + 1 < SB)(lambda: fire_msg(j + 1, ob))
            return carry

        lax.fori_loop(0, SB // 2, pair, 0)
        wait_sc(1)
        plsc.subcore_barrier()
        pltpu.sync_copy(agg_sp.at[pl.ds(row0, NROWS)],
                        agg_o.at[m, pl.ds(row0, NROWS)])


@functools.cache
def _sc_kernels():
    mesh = plsc.VectorSubcoreMesh(core_axis_name="c", subcore_axis_name="s",
                                  num_cores=NC, num_subcores=NS)
    gpos = pl.kernel(
        _sc_gpos,
        out_type=[jax.ShapeDtypeStruct((8, E_PAD), jnp.float32)],
        mesh=mesh,
        compiler_params=pltpu.CompilerParams(needs_layout_passes=False),
        scratch_types=[pltpu.VMEM((N,), jnp.float32),
                       pltpu.VMEM((N,), jnp.float32),
                       pltpu.VMEM((N,), jnp.float32),
                       pltpu.VMEM((GB1, IB), jnp.int32),
                       pltpu.VMEM((GB1, IB), jnp.int32),
                       pltpu.VMEM((8, IB), jnp.float32),
                       pltpu.VMEM((8, IB), jnp.float32),
                       pltpu.SemaphoreType.DMA,
                       pltpu.SemaphoreType.DMA],
    )
    gather1 = pl.kernel(
        _sc_gather1,
        out_type=[jax.ShapeDtypeStruct((E_PAD, C), jnp.float32)],
        mesh=mesh,
        scratch_types=[pltpu.VMEM_SHARED((N, C), jnp.float32),
                       pltpu.VMEM((GB1, IB), jnp.int32),
                       pltpu.VMEM((IB, C), jnp.float32),
                       pltpu.VMEM((IB, C), jnp.float32),
                       pltpu.SemaphoreType.DMA,
                       pltpu.SemaphoreType.DMA,
                       pltpu.SemaphoreType.DMA,
                       pltpu.SemaphoreType.DMA],
    )
    scatter = pl.kernel(
        _sc_scatter,
        out_type=[jax.ShapeDtypeStruct((SH, N, C), jnp.float32)],
        mesh=mesh,
        scratch_types=[pltpu.VMEM_SHARED((N, C), jnp.float32),
                       pltpu.VMEM((SB, IB), jnp.int32),
                       pltpu.VMEM((IB, C), jnp.float32),
                       pltpu.VMEM((IB, C), jnp.float32),
                       pltpu.SemaphoreType.DMA,
                       pltpu.SemaphoreType.DMA,
                       pltpu.SemaphoreType.DMA,
                       pltpu.SemaphoreType.DMA],
    )
    return gpos, gather1, scatter


def _full(shape):
    return pl.BlockSpec(shape, lambda *_: tuple(0 for _ in shape))


def _node_pre(na, we, wup1):
    return pl.pallas_call(
        _node_pre_body,
        out_shape=[jax.ShapeDtypeStruct((N, C), jnp.float32),
                   jax.ShapeDtypeStruct((N, C), jnp.float32)],
    )(na, we, wup1)


def _edge1(pos8, shT, hup1s, rad1):
    eb = lambda w: pl.BlockSpec((BE, w), lambda i: (i, 0))
    return pl.pallas_call(
        _edge1_body,
        grid=(EGRID,),
        in_specs=[pl.BlockSpec((8, BE), lambda i: (0, i)),
                  pl.BlockSpec((4, BE), lambda i: (0, i)), eb(C),
                  _full((NB, 64)), _full((64, 64)), _full((64, 64)),
                  _full((64, 2 * C))],
        out_specs=[pl.BlockSpec((SH, BE, C), lambda i: (0, i, 0)),
                   pl.BlockSpec((NB, BE), lambda i: (0, i)),
                   pl.BlockSpec((4, BE), lambda i: (0, i))],
        out_shape=[jax.ShapeDtypeStruct((SH, E_PAD, C), jnp.float32),
                   jax.ShapeDtypeStruct((NB, E_PAD), jnp.float32),
                   jax.ShapeDtypeStruct((4, E_PAD), jnp.float32)],
    )(pos8, shT, hup1s, *rad1)


def _edge2(efT, yT, hup2s, rad2):
    eb = lambda w: pl.BlockSpec((BE, w), lambda i: (i, 0))
    return pl.pallas_call(
        _edge2_body,
        grid=(EGRID,),
        in_specs=[pl.BlockSpec((NB, BE), lambda i: (0, i)),
                  pl.BlockSpec((4, BE), lambda i: (0, i)), eb(C),
                  _full((NB, 64)), _full((64, 64)), _full((64, 64)),
                  _full((64, 2 * C))],
        out_specs=[pl.BlockSpec((SH, BE, C), lambda i: (0, i, 0))],
        out_shape=[jax.ShapeDtypeStruct((SH, E_PAD, C), jnp.float32)],
    )(efT, yT, hup2s, *rad2)


def _node_mid(agg1, wmix1, na, h0, pc1, wsc1, sc1a, wup2, wr1):
    nb = lambda w: pl.BlockSpec((BN, w), lambda i: (i, 0))
    return pl.pallas_call(
        _node_mid_body,
        grid=(NGRID,),
        in_specs=[pl.BlockSpec((SH, BN, C), lambda i: (0, i, 0)),
                  _full((2, C, C)), nb(NE), nb(C), _full((3, NE, D)),
                  _full((C, D)), _full((NE, 1)), _full((C, C)),
                  _full((C, 1))],
        out_specs=[nb(C), nb(1)],
        out_shape=[jax.ShapeDtypeStruct((N, C), jnp.float32),
                   jax.ShapeDtypeStruct((N, 1), jnp.float32)],
    )(agg1, wmix1, na, h0, pc1, wsc1, sc1a, wup2, wr1)


def _node_out(agg2, wmix2, na, hup2, pc2, wout2, sc2a, nlw1, nlw2, ae, ne1):
    nb = lambda w: pl.BlockSpec((BN, w), lambda i: (i, 0))
    return pl.pallas_call(
        _node_out_body,
        grid=(NGRID,),
        in_specs=[pl.BlockSpec((SH, BN, C), lambda i: (0, i, 0)),
                  _full((2, C, C)), nb(NE), nb(C), _full((3, NE, D)),
                  _full((D, C)), _full((NE, 1)), _full((C, 16)),
                  _full((16, 1)), _full((NE, 1)), nb(1)],
        out_specs=[nb(1)],
        out_shape=[jax.ShapeDtypeStruct((N, 1), jnp.float32)],
    )(agg2, wmix2, na, hup2, pc2, wout2, sc2a, nlw1, nlw2, ae, ne1)


def _graph_sum(ne_rows):
    return pl.pallas_call(
        _graph_sum_body,
        out_shape=jax.ShapeDtypeStruct((1, G), jnp.float32),
    )(ne_rows)


def kernel(positions, node_attrs, shifts, params, edge_index, batch):
    p = params
    src2d = jnp.pad(edge_index[0].astype(jnp.int32),
                    (0, E_PAD - E)).reshape(E_PAD // IB, IB)
    dst2d = jnp.pad(edge_index[1].astype(jnp.int32),
                    (0, E_PAD - E)).reshape(E_PAD // IB, IB)
    shT = jnp.pad(shifts.T, ((0, 1), (0, E_PAD - E)))
    px = jnp.asarray(positions[:, 0])
    py = jnp.asarray(positions[:, 1])
    pz = jnp.asarray(positions[:, 2])
    zeros = jnp.zeros((NROWS, C), jnp.float32)

    gpos, gather1, scatter = _sc_kernels()
    h0, hup1 = _node_pre(node_attrs, p['W_embed'], p['W_up1'])
    pos8, = gpos(px, py, pz, src2d, dst2d)
    hup1s, = gather1(hup1, src2d)
    msg1, efT, yT = _edge1(pos8, shT, hup1s, p['rad1'])
    agg1, = scatter(msg1, dst2d, zeros)
    hup2, ne1 = _node_mid(agg1, p['W_mix1'], node_attrs, h0, p['prod1_c'],
                          p['W_sc1'], p['sc1_a'][:, None], p['W_up2'],
                          p['w_r1'])
    hup2s, = gather1(hup2, src2d)
    msg2, = _edge2(efT, yT, hup2s, p['rad2'])
    agg2, = scatter(msg2, dst2d, zeros)
    ne, = _node_out(agg2, p['W_mix2'], node_attrs, hup2, p['prod2_c'],
                   p['W_out2'], p['sc2_a'][:, None], p['nl_w1'], p['nl_w2'],
                   p['atomic_energies'][:, None], ne1)
    return _graph_sum(ne.reshape(G, N // G)).reshape(G)
